# baseline (speedup 1.0000x reference)
import functools

import jax
import jax.numpy as jnp
from jax import lax
from jax.experimental import pallas as pl
from jax.experimental.pallas import tpu as pltpu
from jax.experimental.pallas import tpu_sc as plsc

VOCAB = 1000000
HALF = VOCAB // 2
D = 64
T = 16384
SCALE = 8.0

_INFO = plsc.get_sparse_core_info()
NC = _INFO.num_cores
NS = _INFO.num_subcores
NW = NC * NS
NT = T // NW
CHUNK = 128
N_CHUNKS = NT // CHUNK

_mesh = plsc.VectorSubcoreMesh(core_axis_name="c", subcore_axis_name="s")


def _make_gather(base):
    @functools.partial(
        pl.kernel,
        mesh=_mesh,
        out_type=jax.ShapeDtypeStruct((T, 2 * D), jnp.float32),
        scratch_types=[
            pltpu.VMEM((NT,), jnp.int32),
            pltpu.VMEM((NT,), jnp.int32),
            pltpu.VMEM((NT, 2 * D), jnp.float32),
            pltpu.SemaphoreType.DMA,
        ],
    )
    def _gather_pairs(table2_hbm, idx_hbm, out_hbm, idx_v, pidx_v, rows_v, sem):
        wid = lax.axis_index("s") * NC + lax.axis_index("c")
        tbase = wid * NT

        pltpu.sync_copy(idx_hbm.at[pl.ds(tbase, NT)], idx_v)

        for j in range(NT // 16):
            sl = pl.ds(j * 16, 16)
            loc = idx_v[sl] - base
            ok = (loc >= 0) & (loc < HALF)
            pidx_v[sl] = jax.lax.shift_right_logical(
                jnp.where(ok, loc, 0), 1
            )

        copies = []
        for j in range(N_CHUNKS):
            copies.append(
                pltpu.async_copy(
                    table2_hbm.at[pidx_v.at[pl.ds(j * CHUNK, CHUNK)]],
                    rows_v.at[pl.ds(j * CHUNK, CHUNK)],
                    sem,
                )
            )
        for c in copies:
            c.wait()

        pltpu.sync_copy(rows_v, out_hbm.at[pl.ds(tbase, NT)])

    return _gather_pairs


_gather_lo = _make_gather(0)
_gather_hi = _make_gather(HALF)


BT = 512


def _select_body(idx_ref, pa_ref, pb_ref, o_ref):
    idx = idx_ref[...]
    in_lo = idx < HALF
    pair_lo = jnp.where(in_lo, pa_ref[:, :D], pb_ref[:, :D])
    pair_hi = jnp.where(in_lo, pa_ref[:, D:], pb_ref[:, D:])
    odd = (idx & 1) == 1
    o_ref[...] = jnp.where(odd, pair_hi, pair_lo) * SCALE


_select = pl.pallas_call(
    _select_body,
    grid=(T // BT,),
    in_specs=[
        pl.BlockSpec((BT, 1), lambda i: (i, 0)),
        pl.BlockSpec((BT, 2 * D), lambda i: (i, 0)),
        pl.BlockSpec((BT, 2 * D), lambda i: (i, 0)),
    ],
    out_specs=pl.BlockSpec((BT, D), lambda i: (i, 0)),
    out_shape=jax.ShapeDtypeStruct((T, D), jnp.float32),
)


def kernel(x, input_embedding_table_VD):
    xi = x.astype(jnp.int32)
    t_lo = input_embedding_table_VD[:HALF].reshape(HALF // 2, 2 * D)
    t_hi = input_embedding_table_VD[HALF:].reshape(HALF // 2, 2 * D)
    pair_lo = _gather_lo(t_lo, xi)
    pair_hi = _gather_hi(t_hi, xi)
    return _select(xi.reshape(T, 1), pair_lo, pair_hi)


# speedup vs baseline: 2.4400x; 2.4400x over previous
import functools

import jax
import jax.numpy as jnp
from jax import lax
from jax.experimental import pallas as pl
from jax.experimental.pallas import tpu as pltpu
from jax.experimental.pallas import tpu_sc as plsc

VOCAB = 1000000
D = 64
T = 16384
SCALE = 8.0

_INFO = plsc.get_sparse_core_info()
NC = _INFO.num_cores
NS = _INFO.num_subcores
NW = NC * NS
NT = T // NW
CHUNK = 128
N_CHUNKS = NT // CHUNK

_mesh = plsc.VectorSubcoreMesh(core_axis_name="c", subcore_axis_name="s")


@functools.partial(
    pl.kernel,
    mesh=_mesh,
    out_type=jax.ShapeDtypeStruct((T, D), jnp.float32),
    scratch_types=[
        pltpu.VMEM((NT,), jnp.int32),
        pltpu.VMEM((NT, D), jnp.float32),
        pltpu.SemaphoreType.DMA,
    ],
    compiler_params=pltpu.CompilerParams(use_tc_tiling_on_sc=False),
)
def _embed(table_hbm, idx_hbm, out_hbm, idx_v, rows_v, sem):
    wid = lax.axis_index("s") * NC + lax.axis_index("c")
    tbase = wid * NT

    pltpu.sync_copy(idx_hbm.at[pl.ds(tbase, NT)], idx_v)

    copies = []
    for j in range(N_CHUNKS):
        copies.append(
            pltpu.async_copy(
                table_hbm.at[idx_v.at[pl.ds(j * CHUNK, CHUNK)]],
                rows_v.at[pl.ds(j * CHUNK, CHUNK)],
                sem,
            )
        )
    for c in copies:
        c.wait()

    def row_body(r, carry):
        for c in range(D // 16):
            sl = pl.ds(c * 16, 16)
            rows_v[r, sl] = rows_v[r, sl] * SCALE
        return carry

    lax.fori_loop(0, NT, row_body, 0, unroll=4)

    pltpu.sync_copy(rows_v, out_hbm.at[pl.ds(tbase, NT)])


def kernel(x, input_embedding_table_VD):
    return _embed(input_embedding_table_VD, x.astype(jnp.int32))
